# baseline (speedup 1.0000x reference)
import jax
import jax.numpy as jnp
from jax.experimental import pallas as pl

N_FEAT = 64
N_CENT = 1024


def _copy_body(c_ref, x_ref, out_ref, loss_ref):
    del c_ref
    out_ref[0] = x_ref[0]
    loss_ref[...] = jnp.zeros_like(loss_ref)


def kernel(x, centroids):
    B = x.shape[0]
    out, loss = pl.pallas_call(
        _copy_body,
        grid=(B,),
        in_specs=[
            pl.BlockSpec((N_FEAT, N_CENT), lambda b: (0, 0)),
            pl.BlockSpec((1, N_FEAT, 32, 32), lambda b: (b, 0, 0, 0)),
        ],
        out_specs=[
            pl.BlockSpec((1, N_FEAT, 32, 32), lambda b: (b, 0, 0, 0)),
            pl.BlockSpec((1, 1), lambda b: (0, 0)),
        ],
        out_shape=[
            jax.ShapeDtypeStruct((B, N_FEAT, 32, 32), jnp.float32),
            jax.ShapeDtypeStruct((1, 1), jnp.float32),
        ],
    )(centroids, x)
    cent_loss = loss[0, 0] / x.size
    return (out, cent_loss)


# speedup vs baseline: 1.2855x; 1.2855x over previous
import jax
import jax.numpy as jnp
from jax.experimental import pallas as pl
from jax.experimental.pallas import tpu as pltpu

N_FEAT = 64
N_CENT = 1024
TOK = 1024
IMGS = 2


def _vq_body(c_ref, x_ref, out_ref, loss_ref, cn_ref):
    b = pl.program_id(0)
    C = c_ref[...]

    @pl.when(b == 0)
    def _():
        ones = jnp.ones((N_FEAT, 1), jnp.float32)
        cn_ref[...] = 0.5 * jax.lax.dot_general(
            C * C, ones, (((0,), (0,)), ((), ())),
            preferred_element_type=jnp.float32)
        loss_ref[...] = jnp.zeros_like(loss_ref)

    hc = cn_ref[...]
    acc = jnp.zeros((), jnp.float32)
    for i in range(IMGS):
        X = x_ref[i]
        s = jax.lax.dot_general(C, X, (((0,), (0,)), ((), ())),
                                preferred_element_type=jnp.float32)
        neg = s - hc
        idx = jnp.argmax(neg, axis=0)
        onehot = (jax.lax.broadcasted_iota(jnp.int32, (N_CENT, TOK), 0)
                  == idx[None, :]).astype(jnp.float32)
        Q = jnp.dot(C, onehot, preferred_element_type=jnp.float32)
        D = Q - X
        out_ref[i] = X + D
        acc = acc + jnp.sum(D * D)

    loss_ref[...] = loss_ref[...] + acc


def kernel(x, centroids):
    B = x.shape[0]
    xr = x.reshape(B, N_FEAT, TOK)
    out, loss = pl.pallas_call(
        _vq_body,
        grid=(B // IMGS,),
        in_specs=[
            pl.BlockSpec((N_FEAT, N_CENT), lambda b: (0, 0)),
            pl.BlockSpec((IMGS, N_FEAT, TOK), lambda b: (b, 0, 0)),
        ],
        out_specs=[
            pl.BlockSpec((IMGS, N_FEAT, TOK), lambda b: (b, 0, 0)),
            pl.BlockSpec((1, 1), lambda b: (0, 0)),
        ],
        out_shape=[
            jax.ShapeDtypeStruct((B, N_FEAT, TOK), jnp.float32),
            jax.ShapeDtypeStruct((1, 1), jnp.float32),
        ],
        scratch_shapes=[pltpu.VMEM((N_CENT, 1), jnp.float32)],
    )(centroids, xr)
    x_quant = out.reshape(x.shape)
    cent_loss = loss[0, 0] / x.size
    return (x_quant, cent_loss)
